# Initial kernel scaffold; baseline (speedup 1.0000x reference)
import jax
import jax.numpy as jnp
from jax.experimental import pallas as pl


def kernel(z, x, mixture_probs):
    raise NotImplementedError("write your pallas kernel here")


# speedup vs baseline: 1.7698x; 1.7698x over previous
import functools

import jax
import jax.numpy as jnp
from jax import lax
from jax.experimental import pallas as pl
from jax.experimental.pallas import tpu as pltpu
from jax.experimental.pallas import tpu_sc as plsc

_N = 1048576
_K = 64
_NC = 2
_NS = 16
_NW = _NC * _NS
_PER_W = _N // _NW
_L = 16

_mesh = plsc.VectorSubcoreMesh(core_axis_name="c", subcore_axis_name="s")


@functools.partial(
    pl.kernel,
    mesh=_mesh,
    compiler_params=pltpu.CompilerParams(needs_layout_passes=False),
    out_type=jax.ShapeDtypeStruct((_N,), jnp.float32),
    scratch_types=[
        pltpu.VMEM((_K,), jnp.float32),
        pltpu.VMEM((_PER_W,), jnp.int32),
        pltpu.VMEM((_PER_W,), jnp.float32),
    ],
)
def _gather_kernel(logp_hbm, z_hbm, out_hbm, table_v, z_v, out_v):
    wid = lax.axis_index("s") * _NC + lax.axis_index("c")
    base = wid * _PER_W
    pltpu.sync_copy(logp_hbm, table_v)
    pltpu.sync_copy(z_hbm.at[pl.ds(base, _PER_W)], z_v)

    def body(i, carry):
        off = i * _L
        idx = z_v[pl.ds(off, _L)]
        out_v[pl.ds(off, _L)] = plsc.load_gather(table_v, [idx])
        return carry

    lax.fori_loop(0, _PER_W // _L, body, 0)
    pltpu.sync_copy(out_v, out_hbm.at[pl.ds(base, _PER_W)])


def kernel(z, x, mixture_probs):
    log_probs = jnp.log(mixture_probs)
    return _gather_kernel(log_probs, z.astype(jnp.int32))


# speedup vs baseline: 2.4244x; 1.3698x over previous
import functools

import jax
import jax.numpy as jnp
from jax import lax
from jax.experimental import pallas as pl
from jax.experimental.pallas import tpu as pltpu
from jax.experimental.pallas import tpu_sc as plsc

_N = 1048576
_K = 64
_NC = 2
_NS = 16
_NW = _NC * _NS
_PER_W = _N // _NW
_L = 16

_mesh = plsc.VectorSubcoreMesh(core_axis_name="c", subcore_axis_name="s")


@functools.partial(
    pl.kernel,
    mesh=_mesh,
    compiler_params=pltpu.CompilerParams(needs_layout_passes=False),
    out_type=jax.ShapeDtypeStruct((_N,), jnp.float32),
    scratch_types=[
        pltpu.VMEM((_K,), jnp.float32),
        pltpu.VMEM((_PER_W,), jnp.int32),
        pltpu.VMEM((_PER_W,), jnp.float32),
    ],
)
def _gather_kernel(logp_hbm, z_hbm, out_hbm, table_v, z_v, out_v):
    wid = lax.axis_index("s") * _NC + lax.axis_index("c")
    base = wid * _PER_W
    pltpu.sync_copy(logp_hbm, table_v)
    pltpu.sync_copy(z_hbm.at[pl.ds(base, _PER_W)], z_v)

    @plsc.parallel_loop(0, _PER_W, _L, unroll=16)
    def _body(off):
        idx = z_v[pl.ds(off, _L)]
        out_v[pl.ds(off, _L)] = plsc.load_gather(table_v, [idx])
    pltpu.sync_copy(out_v, out_hbm.at[pl.ds(base, _PER_W)])


def kernel(z, x, mixture_probs):
    log_probs = jnp.log(mixture_probs)
    return _gather_kernel(log_probs, z.astype(jnp.int32))


# speedup vs baseline: 2.4982x; 1.0305x over previous
import functools
import math

import jax
import jax.numpy as jnp
from jax import lax
from jax.experimental import pallas as pl
from jax.experimental.pallas import tpu as pltpu
from jax.experimental.pallas import tpu_sc as plsc

_N = 1048576
_K = 64
_NC = 2
_NS = 16
_NW = _NC * _NS
_PER_W = _N // _NW
_CHUNK = 8192
_NCHUNK = _PER_W // _CHUNK
_L = 16

_LN2 = math.log(2.0)

_mesh = plsc.VectorSubcoreMesh(core_axis_name="c", subcore_axis_name="s")


@functools.partial(
    pl.kernel,
    mesh=_mesh,
    compiler_params=pltpu.CompilerParams(needs_layout_passes=False),
    out_type=jax.ShapeDtypeStruct((_N,), jnp.float32),
    scratch_types=[
        pltpu.VMEM((_K,), jnp.float32),
        pltpu.VMEM((_K,), jnp.float32),
        pltpu.VMEM((_CHUNK,), jnp.int32),
        pltpu.VMEM((_CHUNK,), jnp.int32),
        pltpu.VMEM((_CHUNK,), jnp.float32),
        pltpu.VMEM((_CHUNK,), jnp.float32),
        pltpu.SemaphoreType.DMA,
        pltpu.SemaphoreType.DMA,
        pltpu.SemaphoreType.DMA,
        pltpu.SemaphoreType.DMA,
        pltpu.SemaphoreType.DMA,
    ],
)
def _gather_kernel(mp_hbm, z_hbm, out_hbm, mp_v, table_v, zb0, zb1, ob0, ob1,
                   sem_t, sem_i0, sem_i1, sem_o0, sem_o1):
    wid = lax.axis_index("s") * _NC + lax.axis_index("c")
    base = wid * _PER_W
    zb = (zb0, zb1)
    ob = (ob0, ob1)
    sem_i = (sem_i0, sem_i1)
    sem_o = (sem_o0, sem_o1)

    t_copy = pltpu.async_copy(mp_hbm, mp_v, sem_t)
    copies_in = [
        pltpu.async_copy(z_hbm.at[pl.ds(base, _CHUNK)], zb[0], sem_i[0])
    ]
    t_copy.wait()

    for k in range(_K // _L):
        y = mp_v[pl.ds(k * _L, _L)]
        bits = lax.bitcast_convert_type(y, jnp.int32)
        w = bits.astype(jnp.float32) * (_LN2 / (1 << 23)) - (127.0 * _LN2)
        for _ in range(3):
            w = w + y * jnp.exp(-w) - 1.0
        table_v[pl.ds(k * _L, _L)] = w

    copies_out = [None] * _NCHUNK
    for c in range(_NCHUNK):
        if c + 1 < _NCHUNK:
            copies_in.append(
                pltpu.async_copy(
                    z_hbm.at[pl.ds(base + (c + 1) * _CHUNK, _CHUNK)],
                    zb[(c + 1) % 2],
                    sem_i[(c + 1) % 2],
                )
            )
        copies_in[c].wait()
        if c >= 2:
            copies_out[c - 2].wait()
        zc = zb[c % 2]
        oc = ob[c % 2]

        @plsc.parallel_loop(0, _CHUNK, _L, unroll=16)
        def _body(off, zc=zc, oc=oc):
            idx = zc[pl.ds(off, _L)]
            oc[pl.ds(off, _L)] = plsc.load_gather(table_v, [idx])

        copies_out[c] = pltpu.async_copy(
            oc, out_hbm.at[pl.ds(base + c * _CHUNK, _CHUNK)], sem_o[c % 2]
        )
    copies_out[_NCHUNK - 2].wait()
    copies_out[_NCHUNK - 1].wait()


def kernel(z, x, mixture_probs):
    return _gather_kernel(mixture_probs, z.astype(jnp.int32))
